# Initial kernel scaffold; baseline (speedup 1.0000x reference)
import jax
import jax.numpy as jnp
from jax.experimental import pallas as pl


def kernel(x, edge_index, W1, b1, W2, b2, W3, b3, Wp1, bp1, Wp2, bp2):
    raise NotImplementedError("write your pallas kernel here")


# speedup vs baseline: 14.1422x; 14.1422x over previous
import functools

import jax
import jax.numpy as jnp
from jax import lax
from jax.experimental import pallas as pl
from jax.experimental.pallas import tpu as pltpu
from jax.experimental.pallas import tpu_sc as plsc

N = 10000
NP = 10240
E = 320000
D = 128
NC = 2
NS = 16
NW = NC * NS
EW = E // NW
C = 128
NFULL = EW // C
CT = EW - NFULL * C
RPT = NP // NS

_f32 = jnp.float32

_mesh = plsc.VectorSubcoreMesh(core_axis_name="c", subcore_axis_name="s")


def _zero_vmem(ref, nrows, width):
    z = jnp.zeros((16,), _f32)

    def body(r, carry):
        for j in range(width // 16):
            ref[r, pl.ds(j * 16, 16)] = z
        return carry

    lax.fori_loop(0, nrows, body, 0)


def _zero_acc_slice(zbuf, acc, r0):
    for t in range(RPT // C):
        pltpu.sync_copy(zbuf, acc.at[pl.ds(r0 + t * C, C)])


@functools.partial(
    pl.kernel,
    out_type=jax.ShapeDtypeStruct((NC, NP, 16), _f32),
    mesh=_mesh,
    scratch_types=[
        pltpu.VMEM_SHARED((NP, 16), _f32),
        pltpu.VMEM((C, 16), _f32),
        pltpu.VMEM((CT, 16), _f32),
        pltpu.VMEM((C,), jnp.int32),
        pltpu.VMEM((CT,), jnp.int32),
    ],
)
def _deg_kernel(dst_hbm, out_hbm, acc, ones_v, ones_t, didx, didx_t):
    c = lax.axis_index("c")
    s = lax.axis_index("s")
    base = (c * NS + s) * EW
    r0 = s * RPT

    _zero_vmem(ones_v, C, 16)
    _zero_acc_slice(ones_v, acc, r0)
    plsc.subcore_barrier()

    one = jnp.ones((16,), _f32)

    def fill(r, carry):
        ones_v[r, :] = one
        return carry

    lax.fori_loop(0, C, fill, 0)

    def fill_t(r, carry):
        ones_t[r, :] = one
        return carry

    lax.fori_loop(0, CT, fill_t, 0)

    def chunk(i, carry):
        cb = base + i * C
        pltpu.sync_copy(dst_hbm.at[pl.ds(cb, C)], didx)
        pltpu.sync_copy(ones_v, acc.at[didx], add=True)
        return carry

    lax.fori_loop(0, NFULL, chunk, 0)
    tb = base + NFULL * C
    pltpu.sync_copy(dst_hbm.at[pl.ds(tb, CT)], didx_t)
    pltpu.sync_copy(ones_t, acc.at[didx_t], add=True)

    plsc.subcore_barrier()
    pltpu.sync_copy(acc.at[pl.ds(r0, RPT)], out_hbm.at[c].at[pl.ds(r0, RPT)])


@functools.partial(
    pl.kernel,
    out_type=jax.ShapeDtypeStruct((NC, NP, D), _f32),
    mesh=_mesh,
    scratch_types=[
        pltpu.VMEM_SHARED((NP, D), _f32),
        pltpu.VMEM((C, D), _f32),
        pltpu.VMEM((CT, D), _f32),
        pltpu.VMEM((C,), jnp.int32),
        pltpu.VMEM((C,), jnp.int32),
        pltpu.VMEM((CT,), jnp.int32),
        pltpu.VMEM((CT,), jnp.int32),
        pltpu.SemaphoreType.DMA,
    ],
)
def _agg_kernel(y_hbm, src_hbm, dst_hbm, out_hbm,
                acc, rows, rows_t, sidx, didx, sidx_t, didx_t, sem):
    c = lax.axis_index("c")
    s = lax.axis_index("s")
    base = (c * NS + s) * EW
    r0 = s * RPT

    _zero_vmem(rows, C, D)
    _zero_acc_slice(rows, acc, r0)
    plsc.subcore_barrier()

    def chunk(i, carry):
        cb = base + i * C
        pltpu.sync_copy(src_hbm.at[pl.ds(cb, C)], sidx)
        pltpu.sync_copy(dst_hbm.at[pl.ds(cb, C)], didx)
        pltpu.async_copy(y_hbm.at[sidx], rows, sem).wait()
        pltpu.sync_copy(rows, acc.at[didx], add=True)
        return carry

    lax.fori_loop(0, NFULL, chunk, 0)
    tb = base + NFULL * C
    pltpu.sync_copy(src_hbm.at[pl.ds(tb, CT)], sidx_t)
    pltpu.sync_copy(dst_hbm.at[pl.ds(tb, CT)], didx_t)
    pltpu.async_copy(y_hbm.at[sidx_t], rows_t, sem).wait()
    pltpu.sync_copy(rows_t, acc.at[didx_t], add=True)

    plsc.subcore_barrier()
    pltpu.sync_copy(acc.at[pl.ds(r0, RPT)], out_hbm.at[c].at[pl.ds(r0, RPT)])


R = 2048
GRID = NP // R


def _dinv_block(dinv16):
    return jnp.broadcast_to(dinv16[:, :1], (R, D))


def _tc1_body(x_ref, w_ref, degp_ref, y_ref, dinv_ref):
    deg = degp_ref[0] + degp_ref[1] + 1.0
    dinv = lax.rsqrt(deg)
    dinv_ref[...] = dinv
    xw = jnp.dot(x_ref[...], w_ref[...], preferred_element_type=_f32)
    y_ref[...] = xw * _dinv_block(dinv)


def _tc1(x, W1, degp):
    return pl.pallas_call(
        _tc1_body,
        grid=(GRID,),
        in_specs=[
            pl.BlockSpec((R, D), lambda i: (i, 0)),
            pl.BlockSpec((D, D), lambda i: (0, 0)),
            pl.BlockSpec((NC, R, 16), lambda i: (0, i, 0)),
        ],
        out_specs=[
            pl.BlockSpec((R, D), lambda i: (i, 0)),
            pl.BlockSpec((R, 16), lambda i: (i, 0)),
        ],
        out_shape=[
            jax.ShapeDtypeStruct((NP, D), _f32),
            jax.ShapeDtypeStruct((NP, 16), _f32),
        ],
    )(x, W1, degp)


def _tc_mid_body(p_ref, y_ref, dinv_ref, b_ref, w_ref, o_ref):
    db = _dinv_block(dinv_ref[...])
    h = jnp.maximum((p_ref[0] + p_ref[1] + y_ref[...]) * db + b_ref[...], 0.0)
    o_ref[...] = jnp.dot(h, w_ref[...], preferred_element_type=_f32) * db


def _tc_mid(p, y, dinv16, b, Wn):
    return pl.pallas_call(
        _tc_mid_body,
        grid=(GRID,),
        in_specs=[
            pl.BlockSpec((NC, R, D), lambda i: (0, i, 0)),
            pl.BlockSpec((R, D), lambda i: (i, 0)),
            pl.BlockSpec((R, 16), lambda i: (i, 0)),
            pl.BlockSpec((1, D), lambda i: (0, 0)),
            pl.BlockSpec((D, D), lambda i: (0, 0)),
        ],
        out_specs=pl.BlockSpec((R, D), lambda i: (i, 0)),
        out_shape=jax.ShapeDtypeStruct((NP, D), _f32),
    )(p, y, dinv16, b, Wn)


def _tc_final_body(p_ref, y_ref, dinv_ref, b_ref,
                   wp1_ref, bp1_ref, wp2_ref, bp2_ref, o_ref):
    db = _dinv_block(dinv_ref[...])
    h = jnp.maximum((p_ref[0] + p_ref[1] + y_ref[...]) * db + b_ref[...], 0.0)
    t = jnp.maximum(
        jnp.dot(h, wp1_ref[...], preferred_element_type=_f32) + bp1_ref[...], 0.0)
    o_ref[...] = jnp.dot(t, wp2_ref[...], preferred_element_type=_f32) + bp2_ref[...]


def _tc_final(p, y, dinv16, b, Wp1, bp1, Wp2, bp2):
    return pl.pallas_call(
        _tc_final_body,
        grid=(GRID,),
        in_specs=[
            pl.BlockSpec((NC, R, D), lambda i: (0, i, 0)),
            pl.BlockSpec((R, D), lambda i: (i, 0)),
            pl.BlockSpec((R, 16), lambda i: (i, 0)),
            pl.BlockSpec((1, D), lambda i: (0, 0)),
            pl.BlockSpec((D, D), lambda i: (0, 0)),
            pl.BlockSpec((1, D), lambda i: (0, 0)),
            pl.BlockSpec((D, D), lambda i: (0, 0)),
            pl.BlockSpec((1, D), lambda i: (0, 0)),
        ],
        out_specs=pl.BlockSpec((R, D), lambda i: (i, 0)),
        out_shape=jax.ShapeDtypeStruct((NP, D), _f32),
    )(p, y, dinv16, b, Wp1, bp1, Wp2, bp2)


def kernel(x, edge_index, W1, b1, W2, b2, W3, b3, Wp1, bp1, Wp2, bp2):
    src = edge_index[0]
    dst = edge_index[1]
    xp = jnp.pad(x, ((0, NP - N), (0, 0)))
    b1r = b1.reshape(1, D)
    b2r = b2.reshape(1, D)
    b3r = b3.reshape(1, D)
    bp1r = bp1.reshape(1, D)
    bp2r = bp2.reshape(1, D)

    degp = _deg_kernel(dst)
    y1, dinv16 = _tc1(xp, W1, degp)
    p1 = _agg_kernel(y1, src, dst)
    y2 = _tc_mid(p1, y1, dinv16, b1r, W2)
    p2 = _agg_kernel(y2, src, dst)
    y3 = _tc_mid(p2, y2, dinv16, b2r, W3)
    p3 = _agg_kernel(y3, src, dst)
    z = _tc_final(p3, y3, dinv16, b3r, Wp1, bp1r, Wp2, bp2r)
    return z[:N]


# speedup vs baseline: 25.5781x; 1.8086x over previous
import functools

import jax
import jax.numpy as jnp
from jax import lax
from jax.experimental import pallas as pl
from jax.experimental.pallas import tpu as pltpu
from jax.experimental.pallas import tpu_sc as plsc

N = 10000
NP = 10240
E = 320000
D = 128
NC = 2
NS = 16
NW = NC * NS
EW = E // NW
C = 128
NFULL = EW // C
CT = EW - NFULL * C
RPT = NP // NS

_f32 = jnp.float32

_mesh = plsc.VectorSubcoreMesh(core_axis_name="c", subcore_axis_name="s")


def _zero_vmem(ref, nrows, width):
    z = jnp.zeros((16,), _f32)

    def body(r, carry):
        for j in range(width // 16):
            ref[r, pl.ds(j * 16, 16)] = z
        return carry

    lax.fori_loop(0, nrows, body, 0)


def _zero_acc_slice(zbuf, acc, r0):
    for t in range(RPT // C):
        pltpu.sync_copy(zbuf, acc.at[pl.ds(r0 + t * C, C)])


@functools.partial(
    pl.kernel,
    out_type=jax.ShapeDtypeStruct((NC, NP, 16), _f32),
    mesh=_mesh,
    scratch_types=[
        pltpu.VMEM_SHARED((NP, 16), _f32),
        pltpu.VMEM((C, 16), _f32),
        pltpu.VMEM((C,), jnp.int32),
        pltpu.VMEM((C,), jnp.int32),
        pltpu.VMEM((CT,), jnp.int32),
        pltpu.SemaphoreType.DMA,
        pltpu.SemaphoreType.DMA,
    ],
)
def _deg_kernel(dstF_hbm, dstT_hbm, out_hbm, acc, ones_v,
                didx0, didx1, didx_t, semi0, semi1):
    c = lax.axis_index("c")
    s = lax.axis_index("s")
    w = c * NS + s
    r0 = s * RPT

    _zero_vmem(ones_v, C, 16)
    _zero_acc_slice(ones_v, acc, r0)
    plsc.subcore_barrier()

    one = jnp.ones((16,), _f32)

    def fill(r, carry):
        ones_v[r, :] = one
        return carry

    lax.fori_loop(0, C, fill, 0)

    didx = (didx0, didx1)
    semi = (semi0, semi1)
    pltpu.async_copy(dstF_hbm.at[w].at[0], didx0, semi0)
    pltpu.async_copy(dstF_hbm.at[w].at[1], didx1, semi1)

    def step(k, p):
        pltpu.make_async_copy(dstF_hbm.at[w].at[k], didx[p], semi[p]).wait()
        pltpu.sync_copy(ones_v, acc.at[didx[p]], add=True)
        pltpu.async_copy(dstF_hbm.at[w].at[k + 2], didx[p], semi[p])

    def body(g, carry):
        step(2 * g, 0)
        step(2 * g + 1, 1)
        return carry

    lax.fori_loop(0, (NFULL - 2) // 2, body, 0)
    pltpu.make_async_copy(dstF_hbm.at[w].at[NFULL - 2], didx0, semi0).wait()
    pltpu.sync_copy(ones_v, acc.at[didx0], add=True)
    pltpu.make_async_copy(dstF_hbm.at[w].at[NFULL - 1], didx1, semi1).wait()
    pltpu.sync_copy(ones_v, acc.at[didx1], add=True)
    pltpu.sync_copy(dstT_hbm.at[w], didx_t)
    pltpu.sync_copy(ones_v.at[pl.ds(0, CT)], acc.at[didx_t], add=True)

    plsc.subcore_barrier()
    pltpu.sync_copy(acc.at[pl.ds(r0, RPT)], out_hbm.at[c].at[pl.ds(r0, RPT)])


@functools.partial(
    pl.kernel,
    out_type=jax.ShapeDtypeStruct((NC, NP, D), _f32),
    mesh=_mesh,
    scratch_types=[
        pltpu.VMEM_SHARED((NP, D), _f32),
        pltpu.VMEM((C, D), _f32),
        pltpu.VMEM((C, D), _f32),
        pltpu.VMEM((C,), jnp.int32),
        pltpu.VMEM((C,), jnp.int32),
        pltpu.VMEM((C,), jnp.int32),
        pltpu.VMEM((C,), jnp.int32),
        pltpu.VMEM((CT,), jnp.int32),
        pltpu.VMEM((CT,), jnp.int32),
        pltpu.SemaphoreType.DMA,
        pltpu.SemaphoreType.DMA,
        pltpu.SemaphoreType.DMA,
        pltpu.SemaphoreType.DMA,
    ],
)
def _agg_kernel(y_hbm, srcF_hbm, dstF_hbm, srcT_hbm, dstT_hbm, out_hbm,
                acc, rows0, rows1, sidx0, didx0, sidx1, didx1,
                sidx_t, didx_t, semi0, semi1, semg0, semg1):
    c = lax.axis_index("c")
    s = lax.axis_index("s")
    w = c * NS + s
    r0 = s * RPT

    _zero_vmem(rows0, C, D)
    _zero_acc_slice(rows0, acc, r0)
    plsc.subcore_barrier()

    sidx = (sidx0, sidx1)
    didx = (didx0, didx1)
    rows = (rows0, rows1)
    semi = (semi0, semi1)
    semg = (semg0, semg1)

    def issue_idx(k, p):
        pltpu.async_copy(srcF_hbm.at[w].at[k], sidx[p], semi[p])
        pltpu.async_copy(dstF_hbm.at[w].at[k], didx[p], semi[p])

    def wait_idx(k, p):
        pltpu.make_async_copy(srcF_hbm.at[w].at[k], sidx[p], semi[p]).wait()
        pltpu.make_async_copy(dstF_hbm.at[w].at[k], didx[p], semi[p]).wait()

    issue_idx(0, 0)
    issue_idx(1, 1)
    wait_idx(0, 0)
    pltpu.async_copy(y_hbm.at[sidx0], rows0, semg0)

    def step(k, p):
        q = 1 - p
        wait_idx(k + 1, q)
        pltpu.async_copy(y_hbm.at[sidx[q]], rows[q], semg[q])
        pltpu.make_async_copy(y_hbm.at[sidx[p]], rows[p], semg[p]).wait()
        pltpu.sync_copy(rows[p], acc.at[didx[p]], add=True)
        issue_idx(k + 2, p)

    def body(g, carry):
        step(2 * g, 0)
        step(2 * g + 1, 1)
        return carry

    lax.fori_loop(0, (NFULL - 2) // 2, body, 0)
    wait_idx(NFULL - 1, 1)
    pltpu.async_copy(y_hbm.at[sidx1], rows1, semg1)
    pltpu.make_async_copy(y_hbm.at[sidx0], rows0, semg0).wait()
    pltpu.sync_copy(rows0, acc.at[didx0], add=True)
    pltpu.make_async_copy(y_hbm.at[sidx1], rows1, semg1).wait()
    pltpu.sync_copy(rows1, acc.at[didx1], add=True)
    pltpu.sync_copy(srcT_hbm.at[w], sidx_t)
    pltpu.sync_copy(dstT_hbm.at[w], didx_t)
    pltpu.async_copy(y_hbm.at[sidx_t], rows0.at[pl.ds(0, CT)], semg0).wait()
    pltpu.sync_copy(rows0.at[pl.ds(0, CT)], acc.at[didx_t], add=True)

    plsc.subcore_barrier()
    pltpu.sync_copy(acc.at[pl.ds(r0, RPT)], out_hbm.at[c].at[pl.ds(r0, RPT)])


R = 2048
GRID = NP // R


def _dinv_block(dinv16):
    return jnp.broadcast_to(dinv16[:, :1], (R, D))


def _tc1_body(x_ref, w_ref, degp_ref, y_ref, dinv_ref):
    deg = degp_ref[0] + degp_ref[1] + 1.0
    dinv = lax.rsqrt(deg)
    dinv_ref[...] = dinv
    xw = jnp.dot(x_ref[...], w_ref[...], preferred_element_type=_f32)
    y_ref[...] = xw * _dinv_block(dinv)


def _tc1(x, W1, degp):
    return pl.pallas_call(
        _tc1_body,
        grid=(GRID,),
        in_specs=[
            pl.BlockSpec((R, D), lambda i: (i, 0)),
            pl.BlockSpec((D, D), lambda i: (0, 0)),
            pl.BlockSpec((NC, R, 16), lambda i: (0, i, 0)),
        ],
        out_specs=[
            pl.BlockSpec((R, D), lambda i: (i, 0)),
            pl.BlockSpec((R, 16), lambda i: (i, 0)),
        ],
        out_shape=[
            jax.ShapeDtypeStruct((NP, D), _f32),
            jax.ShapeDtypeStruct((NP, 16), _f32),
        ],
    )(x, W1, degp)


def _tc_mid_body(p_ref, y_ref, dinv_ref, b_ref, w_ref, o_ref):
    db = _dinv_block(dinv_ref[...])
    h = jnp.maximum((p_ref[0] + p_ref[1] + y_ref[...]) * db + b_ref[...], 0.0)
    o_ref[...] = jnp.dot(h, w_ref[...], preferred_element_type=_f32) * db


def _tc_mid(p, y, dinv16, b, Wn):
    return pl.pallas_call(
        _tc_mid_body,
        grid=(GRID,),
        in_specs=[
            pl.BlockSpec((NC, R, D), lambda i: (0, i, 0)),
            pl.BlockSpec((R, D), lambda i: (i, 0)),
            pl.BlockSpec((R, 16), lambda i: (i, 0)),
            pl.BlockSpec((1, D), lambda i: (0, 0)),
            pl.BlockSpec((D, D), lambda i: (0, 0)),
        ],
        out_specs=pl.BlockSpec((R, D), lambda i: (i, 0)),
        out_shape=jax.ShapeDtypeStruct((NP, D), _f32),
    )(p, y, dinv16, b, Wn)


def _tc_final_body(p_ref, y_ref, dinv_ref, b_ref,
                   wp1_ref, bp1_ref, wp2_ref, bp2_ref, o_ref):
    db = _dinv_block(dinv_ref[...])
    h = jnp.maximum((p_ref[0] + p_ref[1] + y_ref[...]) * db + b_ref[...], 0.0)
    t = jnp.maximum(
        jnp.dot(h, wp1_ref[...], preferred_element_type=_f32) + bp1_ref[...], 0.0)
    o_ref[...] = jnp.dot(t, wp2_ref[...], preferred_element_type=_f32) + bp2_ref[...]


def _tc_final(p, y, dinv16, b, Wp1, bp1, Wp2, bp2):
    return pl.pallas_call(
        _tc_final_body,
        grid=(GRID,),
        in_specs=[
            pl.BlockSpec((NC, R, D), lambda i: (0, i, 0)),
            pl.BlockSpec((R, D), lambda i: (i, 0)),
            pl.BlockSpec((R, 16), lambda i: (i, 0)),
            pl.BlockSpec((1, D), lambda i: (0, 0)),
            pl.BlockSpec((D, D), lambda i: (0, 0)),
            pl.BlockSpec((1, D), lambda i: (0, 0)),
            pl.BlockSpec((D, D), lambda i: (0, 0)),
            pl.BlockSpec((1, D), lambda i: (0, 0)),
        ],
        out_specs=pl.BlockSpec((R, D), lambda i: (i, 0)),
        out_shape=jax.ShapeDtypeStruct((NP, D), _f32),
    )(p, y, dinv16, b, Wp1, bp1, Wp2, bp2)


def kernel(x, edge_index, W1, b1, W2, b2, W3, b3, Wp1, bp1, Wp2, bp2):
    src2 = edge_index[0].reshape(NW, EW)
    dst2 = edge_index[1].reshape(NW, EW)
    srcF = src2[:, :NFULL * C].reshape(NW, NFULL, C)
    dstF = dst2[:, :NFULL * C].reshape(NW, NFULL, C)
    srcT = src2[:, NFULL * C:]
    dstT = dst2[:, NFULL * C:]
    xp = jnp.pad(x, ((0, NP - N), (0, 0)))
    b1r = b1.reshape(1, D)
    b2r = b2.reshape(1, D)
    b3r = b3.reshape(1, D)
    bp1r = bp1.reshape(1, D)
    bp2r = bp2.reshape(1, D)

    degp = _deg_kernel(dstF, dstT)
    y1, dinv16 = _tc1(xp, W1, degp)
    p1 = _agg_kernel(y1, srcF, dstF, srcT, dstT)
    y2 = _tc_mid(p1, y1, dinv16, b1r, W2)
    p2 = _agg_kernel(y2, srcF, dstF, srcT, dstT)
    y3 = _tc_mid(p2, y2, dinv16, b2r, W3)
    p3 = _agg_kernel(y3, srcF, dstF, srcT, dstT)
    z = _tc_final(p3, y3, dinv16, b3r, Wp1, bp1r, Wp2, bp2r)
    return z[:N]


# speedup vs baseline: 28.4855x; 1.1137x over previous
import functools

import jax
import jax.numpy as jnp
from jax import lax
from jax.experimental import pallas as pl
from jax.experimental.pallas import tpu as pltpu
from jax.experimental.pallas import tpu_sc as plsc

N = 10000
NP = 10240
E = 320000
D = 128
NC = 2
NS = 16
NW = NC * NS
EW = E // NW
C = 128
NFULL = EW // C
CT = EW - NFULL * C
RPT = NP // NS

_f32 = jnp.float32

_mesh = plsc.VectorSubcoreMesh(core_axis_name="c", subcore_axis_name="s")


def _zero_vmem(ref, nrows, width):
    z = jnp.zeros((16,), _f32)

    def body(r, carry):
        for j in range(width // 16):
            ref[r, pl.ds(j * 16, 16)] = z
        return carry

    lax.fori_loop(0, nrows, body, 0)


def _zero_acc_slice(zbuf, acc, r0):
    for t in range(RPT // C):
        pltpu.sync_copy(zbuf, acc.at[pl.ds(r0 + t * C, C)])


@functools.partial(
    pl.kernel,
    out_type=jax.ShapeDtypeStruct((NC, NP, 16), _f32),
    mesh=_mesh,
    scratch_types=[
        pltpu.VMEM_SHARED((NP, 16), _f32),
        pltpu.VMEM((C, 16), _f32),
        pltpu.VMEM((C,), jnp.int32),
        pltpu.VMEM((C,), jnp.int32),
        pltpu.VMEM((CT,), jnp.int32),
        pltpu.SemaphoreType.DMA,
        pltpu.SemaphoreType.DMA,
    ],
)
def _deg_kernel(dstF_hbm, dstT_hbm, out_hbm, acc, ones_v,
                didx0, didx1, didx_t, semi0, semi1):
    c = lax.axis_index("c")
    s = lax.axis_index("s")
    w = c * NS + s
    r0 = s * RPT

    _zero_vmem(ones_v, C, 16)
    _zero_acc_slice(ones_v, acc, r0)
    plsc.subcore_barrier()

    one = jnp.ones((16,), _f32)

    def fill(r, carry):
        ones_v[r, :] = one
        return carry

    lax.fori_loop(0, C, fill, 0)

    didx = (didx0, didx1)
    semi = (semi0, semi1)
    pltpu.async_copy(dstF_hbm.at[w].at[0], didx0, semi0)
    pltpu.async_copy(dstF_hbm.at[w].at[1], didx1, semi1)

    def step(k, p):
        pltpu.make_async_copy(dstF_hbm.at[w].at[k], didx[p], semi[p]).wait()
        pltpu.sync_copy(ones_v, acc.at[didx[p]], add=True)
        pltpu.async_copy(dstF_hbm.at[w].at[k + 2], didx[p], semi[p])

    def body(g, carry):
        step(2 * g, 0)
        step(2 * g + 1, 1)
        return carry

    lax.fori_loop(0, (NFULL - 2) // 2, body, 0)
    pltpu.make_async_copy(dstF_hbm.at[w].at[NFULL - 2], didx0, semi0).wait()
    pltpu.sync_copy(ones_v, acc.at[didx0], add=True)
    pltpu.make_async_copy(dstF_hbm.at[w].at[NFULL - 1], didx1, semi1).wait()
    pltpu.sync_copy(ones_v, acc.at[didx1], add=True)
    pltpu.sync_copy(dstT_hbm.at[w], didx_t)
    pltpu.sync_copy(ones_v.at[pl.ds(0, CT)], acc.at[didx_t], add=True)

    plsc.subcore_barrier()
    pltpu.sync_copy(acc.at[pl.ds(r0, RPT)], out_hbm.at[c].at[pl.ds(r0, RPT)])


@functools.partial(
    pl.kernel,
    out_type=jax.ShapeDtypeStruct((NC, NP, D), _f32),
    mesh=_mesh,
    scratch_types=[
        pltpu.VMEM_SHARED((NP, D), _f32),
        pltpu.VMEM((C, D), _f32),
        pltpu.VMEM((C, D), _f32),
        pltpu.VMEM((2, C), jnp.int32),
        pltpu.VMEM((2, C), jnp.int32),
        pltpu.VMEM((2, C), jnp.int32),
        pltpu.VMEM((2, C), jnp.int32),
        pltpu.VMEM((CT,), jnp.int32),
        pltpu.VMEM((CT,), jnp.int32),
        pltpu.SemaphoreType.DMA,
        pltpu.SemaphoreType.DMA,
        pltpu.SemaphoreType.DMA,
        pltpu.SemaphoreType.DMA,
        pltpu.SemaphoreType.DMA,
        pltpu.SemaphoreType.DMA,
        pltpu.SemaphoreType.DMA,
        pltpu.SemaphoreType.DMA,
    ],
)
def _agg_kernel(y_hbm, eiF_hbm, srcT_hbm, dstT_hbm, out_hbm,
                acc, rows0, rows1, sd0, sd1, sd2, sd3, sidx_t, didx_t,
                semi0, semi1, semi2, semi3, semg0, semg1, semsc0, semsc1):
    c = lax.axis_index("c")
    s = lax.axis_index("s")
    w = c * NS + s
    r0 = s * RPT

    sd = (sd0, sd1, sd2, sd3)
    semi = (semi0, semi1, semi2, semi3)
    rows = (rows0, rows1)
    semg = (semg0, semg1)
    semsc = (semsc0, semsc1)

    def issue_idx(k, pi):
        pltpu.async_copy(eiF_hbm.at[w].at[k], sd[pi], semi[pi])

    def wait_idx(k, pi):
        pltpu.make_async_copy(eiF_hbm.at[w].at[k], sd[pi], semi[pi]).wait()

    def start_gather(k, pi, pr):
        pltpu.async_copy(y_hbm.at[sd[pi].at[0]], rows[pr], semg[pr])

    def wait_gather(k, pi, pr):
        pltpu.make_async_copy(y_hbm.at[sd[pi].at[0]], rows[pr], semg[pr]).wait()

    def start_scatter(k, pi, pr):
        pltpu.async_copy(rows[pr], acc.at[sd[pi].at[1]], semsc[pr], add=True)

    def wait_scatter(k, pi, pr):
        pltpu.make_async_copy(rows[pr], acc.at[sd[pi].at[1]], semsc[pr]).wait()

    def full_step(k, pi, pr):
        wait_idx(k + 1, (pi + 1) % 4)
        wait_scatter(k - 1, (pi + 3) % 4, 1 - pr)
        start_gather(k + 1, (pi + 1) % 4, 1 - pr)
        wait_gather(k, pi, pr)
        start_scatter(k, pi, pr)
        issue_idx(k + 3, (pi + 3) % 4)

    issue_idx(0, 0)
    issue_idx(1, 1)
    issue_idx(2, 2)
    _zero_vmem(rows1, C, D)
    _zero_acc_slice(rows1, acc, r0)
    wait_idx(0, 0)
    start_gather(0, 0, 0)
    plsc.subcore_barrier()
    wait_idx(1, 1)
    start_gather(1, 1, 1)
    wait_gather(0, 0, 0)
    start_scatter(0, 0, 0)
    issue_idx(3, 3)

    def body(g, carry):
        k = 4 * g + 1
        full_step(k, 1, 1)
        full_step(k + 1, 2, 0)
        full_step(k + 2, 3, 1)
        full_step(k + 3, 0, 0)
        return carry

    lax.fori_loop(0, 18, body, 0)
    full_step(73, 1, 1)
    full_step(74, 2, 0)
    wait_idx(76, 0)
    wait_scatter(74, 2, 0)
    start_gather(76, 0, 0)
    wait_gather(75, 3, 1)
    start_scatter(75, 3, 1)
    wait_idx(77, 1)
    wait_scatter(75, 3, 1)
    start_gather(77, 1, 1)
    wait_gather(76, 0, 0)
    start_scatter(76, 0, 0)
    wait_scatter(76, 0, 0)
    wait_gather(77, 1, 1)
    start_scatter(77, 1, 1)
    wait_scatter(77, 1, 1)
    pltpu.sync_copy(srcT_hbm.at[w], sidx_t)
    pltpu.sync_copy(dstT_hbm.at[w], didx_t)
    pltpu.async_copy(y_hbm.at[sidx_t], rows1.at[pl.ds(0, CT)], semg1).wait()
    pltpu.sync_copy(rows1.at[pl.ds(0, CT)], acc.at[didx_t], add=True)

    plsc.subcore_barrier()
    pltpu.sync_copy(acc.at[pl.ds(r0, RPT)], out_hbm.at[c].at[pl.ds(r0, RPT)])


R = 2048
GRID = NP // R


def _dinv_block(dinv16):
    return jnp.broadcast_to(dinv16[:, :1], (R, D))


def _tc1_body(x_ref, w_ref, degp_ref, y_ref, dinv_ref):
    deg = degp_ref[0] + degp_ref[1] + 1.0
    dinv = lax.rsqrt(deg)
    dinv_ref[...] = dinv
    xw = jnp.dot(x_ref[...], w_ref[...], preferred_element_type=_f32)
    y_ref[...] = xw * _dinv_block(dinv)


def _tc1(x, W1, degp):
    return pl.pallas_call(
        _tc1_body,
        grid=(GRID,),
        in_specs=[
            pl.BlockSpec((R, D), lambda i: (i, 0)),
            pl.BlockSpec((D, D), lambda i: (0, 0)),
            pl.BlockSpec((NC, R, 16), lambda i: (0, i, 0)),
        ],
        out_specs=[
            pl.BlockSpec((R, D), lambda i: (i, 0)),
            pl.BlockSpec((R, 16), lambda i: (i, 0)),
        ],
        out_shape=[
            jax.ShapeDtypeStruct((NP, D), _f32),
            jax.ShapeDtypeStruct((NP, 16), _f32),
        ],
    )(x, W1, degp)


def _tc_mid_body(p_ref, y_ref, dinv_ref, b_ref, w_ref, o_ref):
    db = _dinv_block(dinv_ref[...])
    h = jnp.maximum((p_ref[0] + p_ref[1] + y_ref[...]) * db + b_ref[...], 0.0)
    o_ref[...] = jnp.dot(h, w_ref[...], preferred_element_type=_f32) * db


def _tc_mid(p, y, dinv16, b, Wn):
    return pl.pallas_call(
        _tc_mid_body,
        grid=(GRID,),
        in_specs=[
            pl.BlockSpec((NC, R, D), lambda i: (0, i, 0)),
            pl.BlockSpec((R, D), lambda i: (i, 0)),
            pl.BlockSpec((R, 16), lambda i: (i, 0)),
            pl.BlockSpec((1, D), lambda i: (0, 0)),
            pl.BlockSpec((D, D), lambda i: (0, 0)),
        ],
        out_specs=pl.BlockSpec((R, D), lambda i: (i, 0)),
        out_shape=jax.ShapeDtypeStruct((NP, D), _f32),
    )(p, y, dinv16, b, Wn)


def _tc_final_body(p_ref, y_ref, dinv_ref, b_ref,
                   wp1_ref, bp1_ref, wp2_ref, bp2_ref, o_ref):
    db = _dinv_block(dinv_ref[...])
    h = jnp.maximum((p_ref[0] + p_ref[1] + y_ref[...]) * db + b_ref[...], 0.0)
    t = jnp.maximum(
        jnp.dot(h, wp1_ref[...], preferred_element_type=_f32) + bp1_ref[...], 0.0)
    o_ref[...] = jnp.dot(t, wp2_ref[...], preferred_element_type=_f32) + bp2_ref[...]


def _tc_final(p, y, dinv16, b, Wp1, bp1, Wp2, bp2):
    return pl.pallas_call(
        _tc_final_body,
        grid=(GRID,),
        in_specs=[
            pl.BlockSpec((NC, R, D), lambda i: (0, i, 0)),
            pl.BlockSpec((R, D), lambda i: (i, 0)),
            pl.BlockSpec((R, 16), lambda i: (i, 0)),
            pl.BlockSpec((1, D), lambda i: (0, 0)),
            pl.BlockSpec((D, D), lambda i: (0, 0)),
            pl.BlockSpec((1, D), lambda i: (0, 0)),
            pl.BlockSpec((D, D), lambda i: (0, 0)),
            pl.BlockSpec((1, D), lambda i: (0, 0)),
        ],
        out_specs=pl.BlockSpec((R, D), lambda i: (i, 0)),
        out_shape=jax.ShapeDtypeStruct((NP, D), _f32),
    )(p, y, dinv16, b, Wp1, bp1, Wp2, bp2)


def kernel(x, edge_index, W1, b1, W2, b2, W3, b3, Wp1, bp1, Wp2, bp2):
    src2 = edge_index[0].reshape(NW, EW)
    dst2 = edge_index[1].reshape(NW, EW)
    srcF = src2[:, :NFULL * C].reshape(NW, NFULL, C)
    dstF = dst2[:, :NFULL * C].reshape(NW, NFULL, C)
    srcT = src2[:, NFULL * C:]
    dstT = dst2[:, NFULL * C:]
    xp = jnp.pad(x, ((0, NP - N), (0, 0)))
    b1r = b1.reshape(1, D)
    b2r = b2.reshape(1, D)
    b3r = b3.reshape(1, D)
    bp1r = bp1.reshape(1, D)
    bp2r = bp2.reshape(1, D)

    eiF = jnp.stack([srcF, dstF], axis=2)

    degp = _deg_kernel(dstF, dstT)
    y1, dinv16 = _tc1(xp, W1, degp)
    p1 = _agg_kernel(y1, eiF, srcT, dstT)
    y2 = _tc_mid(p1, y1, dinv16, b1r, W2)
    p2 = _agg_kernel(y2, eiF, srcT, dstT)
    y3 = _tc_mid(p2, y2, dinv16, b2r, W3)
    p3 = _agg_kernel(y3, eiF, srcT, dstT)
    z = _tc_final(p3, y3, dinv16, b3r, Wp1, bp1r, Wp2, bp2r)
    return z[:N]
